# baseline (speedup 1.0000x reference)
import functools

import jax
import jax.numpy as jnp
from jax.experimental import pallas as pl


N = 4096
BM = 256


def _mm_kernel(a_ref, x_ref, o_ref):
    o_ref[...] = jnp.dot(a_ref[...], x_ref[...],
                         preferred_element_type=jnp.float32).astype(o_ref.dtype)


def _mm_w_kernel(a_ref, x_ref, w_ref, o_ref):
    t = jnp.dot(a_ref[...], x_ref[...], preferred_element_type=jnp.float32)
    o_ref[...] = jnp.dot(t, w_ref[...], preferred_element_type=jnp.float32)


@functools.partial(jax.jit, static_argnames=("out_dtype",))
def _mm(mat, x, out_dtype=jnp.float32):
    n, f = x.shape
    return pl.pallas_call(
        _mm_kernel,
        grid=(n // BM,),
        in_specs=[
            pl.BlockSpec((BM, n), lambda i: (i, 0)),
            pl.BlockSpec((n, f), lambda i: (0, 0)),
        ],
        out_specs=pl.BlockSpec((BM, f), lambda i: (i, 0)),
        out_shape=jax.ShapeDtypeStruct((n, f), out_dtype),
    )(mat, x)


@functools.partial(jax.jit, static_argnames=())
def _mm_w(mat, x, w):
    n, f = x.shape
    emb = w.shape[1]
    return pl.pallas_call(
        _mm_w_kernel,
        grid=(n // BM,),
        in_specs=[
            pl.BlockSpec((BM, n), lambda i: (i, 0)),
            pl.BlockSpec((n, f), lambda i: (0, 0)),
            pl.BlockSpec((f, emb), lambda i: (0, 0)),
        ],
        out_specs=pl.BlockSpec((BM, emb), lambda i: (i, 0)),
        out_shape=jax.ShapeDtypeStruct((n, emb), jnp.float32),
    )(mat, x, w)


def kernel(feature, adj_self, adj_dele, weight):
    a16 = adj_self.astype(jnp.bfloat16)
    b16 = adj_dele.astype(jnp.bfloat16)
    t = _mm(b16, feature.astype(jnp.bfloat16), jnp.bfloat16)
    t = _mm(a16, t, jnp.bfloat16)
    t = _mm(b16, t, jnp.bfloat16)
    return _mm_w(a16, t, weight)


# speedup vs baseline: 1.3610x; 1.3610x over previous
import functools

import jax
import jax.numpy as jnp
from jax.experimental import pallas as pl


N = 4096
BM = 256


def _mm_kernel(a_ref, x_ref, o_ref):
    a = a_ref[...].astype(jnp.bfloat16)
    x = x_ref[...].astype(jnp.bfloat16)
    o_ref[...] = jnp.dot(a, x, preferred_element_type=jnp.float32).astype(o_ref.dtype)


def _mm_w_kernel(a_ref, x_ref, w_ref, o_ref):
    a = a_ref[...].astype(jnp.bfloat16)
    x = x_ref[...].astype(jnp.bfloat16)
    t = jnp.dot(a, x, preferred_element_type=jnp.float32)
    o_ref[...] = jnp.dot(t, w_ref[...], preferred_element_type=jnp.float32)


@functools.partial(jax.jit, static_argnames=("out_dtype",))
def _mm(mat, x, out_dtype=jnp.float32):
    n, f = x.shape
    return pl.pallas_call(
        _mm_kernel,
        grid=(n // BM,),
        in_specs=[
            pl.BlockSpec((BM, n), lambda i: (i, 0)),
            pl.BlockSpec((n, f), lambda i: (0, 0)),
        ],
        out_specs=pl.BlockSpec((BM, f), lambda i: (i, 0)),
        out_shape=jax.ShapeDtypeStruct((n, f), out_dtype),
    )(mat, x)


@functools.partial(jax.jit, static_argnames=())
def _mm_w(mat, x, w):
    n, f = x.shape
    emb = w.shape[1]
    return pl.pallas_call(
        _mm_w_kernel,
        grid=(n // BM,),
        in_specs=[
            pl.BlockSpec((BM, n), lambda i: (i, 0)),
            pl.BlockSpec((n, f), lambda i: (0, 0)),
            pl.BlockSpec((f, emb), lambda i: (0, 0)),
        ],
        out_specs=pl.BlockSpec((BM, emb), lambda i: (i, 0)),
        out_shape=jax.ShapeDtypeStruct((n, emb), jnp.float32),
    )(mat, x, w)


def kernel(feature, adj_self, adj_dele, weight):
    t = _mm(adj_dele, feature)
    t = _mm(adj_self, t)
    t = _mm(adj_dele, t)
    return _mm_w(adj_self, t, weight)
